# baseline (speedup 1.0000x reference)
import functools

import jax
import jax.numpy as jnp
from jax import lax
from jax.experimental import pallas as pl
from jax.experimental.pallas import tpu as pltpu
from jax.experimental.pallas import tpu_sc as plsc

N = 6_400_000
S = 4096
NW = 32
PW = N // NW
C = 20_000
NCHUNK = PW // C
LANES = 16
LBLOCK = C // LANES
RAIL = LBLOCK // 2

_mesh = plsc.VectorSubcoreMesh(core_axis_name="c", subcore_axis_name="s")


@functools.partial(
    pl.kernel,
    out_type=jax.ShapeDtypeStruct((NW, S), jnp.float32),
    mesh=_mesh,
    compiler_params=pltpu.CompilerParams(needs_layout_passes=False,
                                         use_tc_tiling_on_sc=False),
    scratch_types=[
        pltpu.VMEM((C,), jnp.float32),
        pltpu.VMEM((C,), jnp.float32),
        pltpu.VMEM((C,), jnp.int32),
        pltpu.VMEM((C,), jnp.int32),
        pltpu.VMEM((S,), jnp.float32),
        pltpu.SemaphoreType.DMA,
        pltpu.SemaphoreType.DMA,
    ],
)
def _sc_segsum(e_hbm, b_hbm, out_hbm, e0, e1, b0, b1, accum, sem0, sem1):
    cid = lax.axis_index("c")
    sid = lax.axis_index("s")
    wid = cid * 16 + sid
    base = wid * PW

    ebufs = (e0, e1)
    bbufs = (b0, b1)
    sems = (sem0, sem1)

    def _issue(k):
        slot = k % 2
        start = base + k * C
        ce = pltpu.async_copy(e_hbm.at[pl.ds(start, C)], ebufs[slot], sems[slot])
        cb = pltpu.async_copy(b_hbm.at[pl.ds(start, C)], bbufs[slot], sems[slot])
        return ce, cb

    pending = {0: _issue(0), 1: _issue(1)}

    zeros16 = jnp.zeros((LANES,), jnp.float32)

    def _zero(i, carry):
        accum[pl.ds(i * LANES, LANES)] = zeros16
        return carry

    lax.fori_loop(0, S // LANES, _zero, 0)

    iota16 = lax.iota(jnp.int32, LANES)
    rail0 = iota16 * LBLOCK
    rail1 = rail0 + RAIL
    zero_r = jnp.zeros((LANES,), jnp.float32)

    for k in range(NCHUNK):
        slot = k % 2
        ce, cb = pending.pop(k)
        ce.wait()
        cb.wait()
        eb = ebufs[slot]
        bb = bbufs[slot]

        ca0 = plsc.load_gather(bb, [rail0])
        cb0 = plsc.load_gather(bb, [rail1])

        def _step(j, carry):
            cur_a, run_a, cur_b, run_b = carry
            ia = rail0 + j
            ib = rail1 + j
            xa = plsc.load_gather(eb, [ia])
            ba = plsc.load_gather(bb, [ia])
            xb = plsc.load_gather(eb, [ib])
            bv = plsc.load_gather(bb, [ib])
            fa = ba != cur_a
            fb = bv != cur_b
            plsc.addupdate_scatter(accum, [cur_a], run_a, mask=fa)
            plsc.addupdate_scatter(accum, [cur_b], run_b, mask=fb)
            run_a = jnp.where(fa, xa, run_a + xa)
            run_b = jnp.where(fb, xb, run_b + xb)
            return ba, run_a, bv, run_b

        ca_f, ra_f, cb_f, rb_f = plsc.parallel_loop(
            0, RAIL, carry=(ca0, zero_r, cb0, zero_r), unroll=5)(_step)
        plsc.addupdate_scatter(accum, [ca_f], ra_f)
        plsc.addupdate_scatter(accum, [cb_f], rb_f)

        if k + 2 < NCHUNK:
            pending[k + 2] = _issue(k + 2)

    pltpu.sync_copy(accum, out_hbm.at[wid])


def _combine_body(p_ref, o_ref):
    o_ref[...] = jnp.sum(p_ref[...], axis=0, keepdims=True)


def kernel(atomic_energy, batch):
    e = atomic_energy.reshape(N)
    b = batch.astype(jnp.int32)
    partials = _sc_segsum(e, b)
    out = pl.pallas_call(
        _combine_body,
        out_shape=jax.ShapeDtypeStruct((1, S), jnp.float32),
    )(partials)
    return out.reshape(S, 1)


# speedup vs baseline: 1.0439x; 1.0439x over previous
import functools

import jax
import jax.numpy as jnp
from jax import lax
from jax.experimental import pallas as pl
from jax.experimental.pallas import tpu as pltpu
from jax.experimental.pallas import tpu_sc as plsc

N = 6_400_000
S = 4096
NW = 32
PW = N // NW
C = 20_000
NCHUNK = PW // C
LANES = 16
LBLOCK = C // LANES
RAIL = LBLOCK // 2

_mesh = plsc.VectorSubcoreMesh(core_axis_name="c", subcore_axis_name="s")


@functools.partial(
    pl.kernel,
    out_type=jax.ShapeDtypeStruct((NW, S), jnp.float32),
    mesh=_mesh,
    compiler_params=pltpu.CompilerParams(needs_layout_passes=False),
    scratch_types=[
        pltpu.VMEM((C,), jnp.float32),
        pltpu.VMEM((C,), jnp.float32),
        pltpu.VMEM((C,), jnp.int32),
        pltpu.VMEM((C,), jnp.int32),
        pltpu.VMEM((S,), jnp.float32),
        pltpu.SemaphoreType.DMA,
        pltpu.SemaphoreType.DMA,
    ],
)
def _sc_segsum(e_hbm, b_hbm, out_hbm, e0, e1, b0, b1, accum, sem0, sem1):
    cid = lax.axis_index("c")
    sid = lax.axis_index("s")
    wid = cid * 16 + sid
    base = wid * PW

    ebufs = (e0, e1)
    bbufs = (b0, b1)
    sems = (sem0, sem1)

    def _issue(k):
        slot = k % 2
        start = base + k * C
        ce = pltpu.async_copy(e_hbm.at[pl.ds(start, C)], ebufs[slot], sems[slot])
        cb = pltpu.async_copy(b_hbm.at[pl.ds(start, C)], bbufs[slot], sems[slot])
        return ce, cb

    pending = {0: _issue(0), 1: _issue(1)}

    zeros16 = jnp.zeros((LANES,), jnp.float32)

    def _zero(i, carry):
        accum[pl.ds(i * LANES, LANES)] = zeros16
        return carry

    lax.fori_loop(0, S // LANES, _zero, 0)

    iota16 = lax.iota(jnp.int32, LANES)
    rail0 = iota16 * LBLOCK
    rail1 = rail0 + RAIL
    zero_r = jnp.zeros((LANES,), jnp.float32)

    for k in range(NCHUNK):
        slot = k % 2
        ce, cb = pending.pop(k)
        ce.wait()
        cb.wait()
        eb = ebufs[slot]
        bb = bbufs[slot]

        ca0 = plsc.load_gather(bb, [rail0])
        cb0 = plsc.load_gather(bb, [rail1])

        def _step(j, carry):
            cur_a, run_a, cur_b, run_b = carry
            ia = rail0 + j
            ib = rail1 + j
            xa = plsc.load_gather(eb, [ia])
            ba = plsc.load_gather(bb, [ia])
            xb = plsc.load_gather(eb, [ib])
            bv = plsc.load_gather(bb, [ib])
            fa = ba != cur_a
            fb = bv != cur_b
            plsc.addupdate_scatter(accum, [cur_a], run_a, mask=fa)
            plsc.addupdate_scatter(accum, [cur_b], run_b, mask=fb)
            run_a = jnp.where(fa, xa, run_a + xa)
            run_b = jnp.where(fb, xb, run_b + xb)
            return ba, run_a, bv, run_b

        ca_f, ra_f, cb_f, rb_f = plsc.parallel_loop(
            0, RAIL, carry=(ca0, zero_r, cb0, zero_r), unroll=5)(_step)
        plsc.addupdate_scatter(accum, [ca_f], ra_f)
        plsc.addupdate_scatter(accum, [cb_f], rb_f)

        if k + 2 < NCHUNK:
            pending[k + 2] = _issue(k + 2)

    pltpu.sync_copy(accum, out_hbm.at[wid])


def _combine_body(p_ref, o_ref):
    o_ref[...] = jnp.sum(p_ref[...], axis=0, keepdims=True)


def kernel(atomic_energy, batch):
    e = atomic_energy.reshape(N)
    b = batch.astype(jnp.int32)
    partials = _sc_segsum(e, b)
    out = pl.pallas_call(
        _combine_body,
        out_shape=jax.ShapeDtypeStruct((1, S), jnp.float32),
    )(partials)
    return out.reshape(S, 1)
